# baseline (speedup 1.0000x reference)
import functools
import math

import jax
import jax.numpy as jnp
from jax import lax
from jax.experimental import pallas as pl
from jax.experimental.pallas import tpu as pltpu
from jax.experimental.pallas import tpu_sc as plsc

D_MODEL = 64
SCALE = math.sqrt(D_MODEL)
NUM_CORES = 2
NUM_SUBCORES = 16
NUM_WORKERS = NUM_CORES * NUM_SUBCORES
LANES = 16
SEQ = 200
SEQ_PAD = 256
NBUF = 2


def _emb_body(x_hbm, table_hbm, out_hbm, *scratch, rows_per_tile):
    idx_v = scratch[:NBUF]
    rows_v = scratch[NBUF:2 * NBUF]
    gsem = scratch[2 * NBUF:3 * NBUF]
    ssem = scratch[3 * NBUF:4 * NBUF]

    wid = lax.axis_index("s") * NUM_CORES + lax.axis_index("c")
    r0 = wid * rows_per_tile

    for b in range(NBUF):
        pltpu.sync_copy(x_hbm.at[pl.ds((r0 + b) * SEQ_PAD, SEQ_PAD)],
                        idx_v[b])
        pltpu.async_copy(table_hbm.at[idx_v[b]], rows_v[b], gsem[b])

    def super_body(k, carry):
        for b in range(NBUF):
            cur = k * NBUF + b
            r = r0 + cur
            pltpu.make_async_copy(table_hbm.at[idx_v[b]], rows_v[b],
                                  gsem[b]).wait()

            @plsc.parallel_loop(0, SEQ, step=1, unroll=8)
            def _mul(i):
                for j in range(D_MODEL // LANES):
                    sl = pl.ds(j * LANES, LANES)
                    rows_v[b][i, sl] = rows_v[b][i, sl] * SCALE

            pltpu.async_copy(rows_v[b].at[pl.ds(0, SEQ)],
                             out_hbm.at[pl.ds(r * SEQ, SEQ)], ssem[b])
            nxt = cur + NBUF

            @pl.when(nxt < rows_per_tile)
            def _():
                pltpu.sync_copy(
                    x_hbm.at[pl.ds((r0 + nxt) * SEQ_PAD, SEQ_PAD)], idx_v[b])
                pltpu.make_async_copy(
                    rows_v[b].at[pl.ds(0, SEQ)],
                    out_hbm.at[pl.ds(r * SEQ, SEQ)], ssem[b]).wait()
                pltpu.async_copy(table_hbm.at[idx_v[b]], rows_v[b], gsem[b])

        return carry

    lax.fori_loop(0, rows_per_tile // NBUF, super_body, 0)

    for b in range(NBUF):
        r = r0 + rows_per_tile - NBUF + b
        pltpu.make_async_copy(rows_v[b].at[pl.ds(0, SEQ)],
                              out_hbm.at[pl.ds(r * SEQ, SEQ)], ssem[b]).wait()


def kernel(x, table):
    n_rows, seq = x.shape
    assert seq == SEQ and n_rows % (NUM_WORKERS * NBUF) == 0
    rows_per_tile = n_rows // NUM_WORKERS
    xp = jnp.pad(x, ((0, 0), (0, SEQ_PAD - SEQ))).reshape(n_rows * SEQ_PAD)

    mesh = plsc.VectorSubcoreMesh(
        core_axis_name="c", subcore_axis_name="s",
        num_cores=NUM_CORES, num_subcores=NUM_SUBCORES,
    )
    scratch = (
        [pltpu.VMEM((SEQ_PAD,), jnp.int32) for _ in range(NBUF)]
        + [pltpu.VMEM((SEQ_PAD, D_MODEL), jnp.float32) for _ in range(NBUF)]
        + [pltpu.SemaphoreType.DMA for _ in range(2 * NBUF)]
    )
    f = functools.partial(
        pl.kernel,
        out_type=jax.ShapeDtypeStruct((n_rows * SEQ, D_MODEL), jnp.float32),
        mesh=mesh,
        scratch_types=scratch,
        compiler_params=pltpu.CompilerParams(use_tc_tiling_on_sc=False),
    )(functools.partial(_emb_body, rows_per_tile=rows_per_tile))
    out = f(xp, table)
    return out.reshape(n_rows, SEQ, D_MODEL)


# speedup vs baseline: 4.6296x; 4.6296x over previous
import functools
import math

import jax
import jax.numpy as jnp
from jax import lax
from jax.experimental import pallas as pl
from jax.experimental.pallas import tpu as pltpu
from jax.experimental.pallas import tpu_sc as plsc

D_MODEL = 64
SCALE = math.sqrt(D_MODEL)
NUM_CORES = 2
NUM_SUBCORES = 16
NUM_WORKERS = NUM_CORES * NUM_SUBCORES
LANES = 16
SEQ = 200
SEQ_PAD = 256
NBUF = 2


def _emb_body(x_hbm, table_hbm, out_hbm, *scratch, rows_per_tile):
    idx_v = scratch[:NBUF]
    rows_v = scratch[NBUF:2 * NBUF]
    gsem = scratch[2 * NBUF:3 * NBUF]
    ssem = scratch[3 * NBUF:4 * NBUF]

    wid = lax.axis_index("s") * NUM_CORES + lax.axis_index("c")
    r0 = wid * rows_per_tile

    for b in range(NBUF):
        pltpu.sync_copy(x_hbm.at[pl.ds((r0 + b) * SEQ_PAD, SEQ_PAD)],
                        idx_v[b])
        pltpu.async_copy(table_hbm.at[idx_v[b]], rows_v[b], gsem[b])

    def super_body(k, carry):
        for b in range(NBUF):
            cur = k * NBUF + b
            r = r0 + cur
            pltpu.make_async_copy(table_hbm.at[idx_v[b]], rows_v[b],
                                  gsem[b]).wait()

            @plsc.parallel_loop(0, SEQ, step=1, unroll=8)
            def _mul(i):
                for j in range(D_MODEL // LANES):
                    sl = pl.ds(j * LANES, LANES)
                    rows_v[b][i, sl] = rows_v[b][i, sl] * SCALE

            pltpu.async_copy(rows_v[b].at[pl.ds(0, SEQ)],
                             out_hbm.at[pl.ds(r * SEQ, SEQ)], ssem[b])
            nxt = cur + NBUF

            @pl.when(nxt < rows_per_tile)
            def _():
                pltpu.sync_copy(
                    x_hbm.at[pl.ds((r0 + nxt) * SEQ_PAD, SEQ_PAD)], idx_v[b])
                pltpu.make_async_copy(
                    rows_v[b].at[pl.ds(0, SEQ)],
                    out_hbm.at[pl.ds(r * SEQ, SEQ)], ssem[b]).wait()
                pltpu.async_copy(table_hbm.at[idx_v[b]], rows_v[b], gsem[b])

        return carry

    lax.fori_loop(0, rows_per_tile // NBUF, super_body, 0)

    for b in range(NBUF):
        r = r0 + rows_per_tile - NBUF + b
        pltpu.make_async_copy(rows_v[b].at[pl.ds(0, SEQ)],
                              out_hbm.at[pl.ds(r * SEQ, SEQ)], ssem[b]).wait()


def kernel(x, table):
    n_rows, seq = x.shape
    assert seq == SEQ and n_rows % (NUM_WORKERS * NBUF) == 0
    rows_per_tile = n_rows // NUM_WORKERS
    filler = (jnp.arange(n_rows * (SEQ_PAD - SEQ), dtype=jnp.int32)
              .reshape(n_rows, SEQ_PAD - SEQ) * 4093) % table.shape[0]
    xp = jnp.concatenate([x, filler], axis=1).reshape(n_rows * SEQ_PAD)

    mesh = plsc.VectorSubcoreMesh(
        core_axis_name="c", subcore_axis_name="s",
        num_cores=NUM_CORES, num_subcores=NUM_SUBCORES,
    )
    scratch = (
        [pltpu.VMEM((SEQ_PAD,), jnp.int32) for _ in range(NBUF)]
        + [pltpu.VMEM((SEQ_PAD, D_MODEL), jnp.float32) for _ in range(NBUF)]
        + [pltpu.SemaphoreType.DMA for _ in range(2 * NBUF)]
    )
    f = functools.partial(
        pl.kernel,
        out_type=jax.ShapeDtypeStruct((n_rows * SEQ, D_MODEL), jnp.float32),
        mesh=mesh,
        scratch_types=scratch,
        compiler_params=pltpu.CompilerParams(use_tc_tiling_on_sc=False),
    )(functools.partial(_emb_body, rows_per_tile=rows_per_tile))
    out = f(xp, table)
    return out.reshape(n_rows, SEQ, D_MODEL)


# speedup vs baseline: 4.6969x; 1.0145x over previous
import functools
import math

import jax
import jax.numpy as jnp
from jax import lax
from jax.experimental import pallas as pl
from jax.experimental.pallas import tpu as pltpu
from jax.experimental.pallas import tpu_sc as plsc

D_MODEL = 64
SCALE = math.sqrt(D_MODEL)
NUM_CORES = 2
NUM_SUBCORES = 16
NUM_WORKERS = NUM_CORES * NUM_SUBCORES
LANES = 16
SEQ = 200
NBUF = 2


def _emb_body(x_hbm, table_hbm, out_hbm, *scratch, rows_per_tile):
    idx_v = scratch[:NBUF]
    rows_v = scratch[NBUF:2 * NBUF]
    gsem = scratch[2 * NBUF:3 * NBUF]
    ssem = scratch[3 * NBUF:4 * NBUF]

    wid = lax.axis_index("s") * NUM_CORES + lax.axis_index("c")
    r0 = wid * rows_per_tile

    for b in range(NBUF):
        pltpu.sync_copy(x_hbm.at[r0 + b], idx_v[b])
        pltpu.async_copy(table_hbm.at[idx_v[b]], rows_v[b], gsem[b])

    def super_body(k, carry):
        for b in range(NBUF):
            cur = k * NBUF + b
            r = r0 + cur
            pltpu.make_async_copy(table_hbm.at[idx_v[b]], rows_v[b],
                                  gsem[b]).wait()

            @plsc.parallel_loop(0, SEQ, step=1, unroll=8)
            def _mul(i):
                for j in range(D_MODEL // LANES):
                    sl = pl.ds(j * LANES, LANES)
                    rows_v[b][i, sl] = rows_v[b][i, sl] * SCALE

            pltpu.async_copy(rows_v[b], out_hbm.at[r], ssem[b])
            nxt = cur + NBUF

            @pl.when(nxt < rows_per_tile)
            def _():
                pltpu.sync_copy(x_hbm.at[r0 + nxt], idx_v[b])
                pltpu.make_async_copy(rows_v[b], out_hbm.at[r],
                                      ssem[b]).wait()
                pltpu.async_copy(table_hbm.at[idx_v[b]], rows_v[b], gsem[b])

        return carry

    lax.fori_loop(0, rows_per_tile // NBUF, super_body, 0)

    for b in range(NBUF):
        r = r0 + rows_per_tile - NBUF + b
        pltpu.make_async_copy(rows_v[b], out_hbm.at[r], ssem[b]).wait()


def kernel(x, table):
    n_rows, seq = x.shape
    assert seq == SEQ and n_rows % (NUM_WORKERS * NBUF) == 0
    rows_per_tile = n_rows // NUM_WORKERS

    mesh = plsc.VectorSubcoreMesh(
        core_axis_name="c", subcore_axis_name="s",
        num_cores=NUM_CORES, num_subcores=NUM_SUBCORES,
    )
    scratch = (
        [pltpu.VMEM((SEQ,), jnp.int32) for _ in range(NBUF)]
        + [pltpu.VMEM((SEQ, D_MODEL), jnp.float32) for _ in range(NBUF)]
        + [pltpu.SemaphoreType.DMA for _ in range(2 * NBUF)]
    )
    f = functools.partial(
        pl.kernel,
        out_type=jax.ShapeDtypeStruct((n_rows, SEQ, D_MODEL), jnp.float32),
        mesh=mesh,
        scratch_types=scratch,
        compiler_params=pltpu.CompilerParams(use_tc_tiling_on_sc=False),
    )(functools.partial(_emb_body, rows_per_tile=rows_per_tile))
    return f(x, table)
